# baseline (speedup 1.0000x reference)
import functools

import jax
import jax.numpy as jnp
from jax import lax
from jax.experimental import pallas as pl
from jax.experimental.pallas import tpu as pltpu
from jax.experimental.pallas import tpu_sc as plsc

HIDDEN = 128
SEQ = 200
L = 16
NSEG = HIDDEN // L
NW = 32
CHUNK = 100


@functools.lru_cache(maxsize=None)
def _make_sc_kernel(batch: int):
    n_tokens = batch * SEQ
    assert n_tokens % (NW * SEQ) == 0
    tok_per_w = n_tokens // NW
    chunks_per_w = tok_per_w // CHUNK
    assert chunks_per_w % 2 == 0
    groups = chunks_per_w // 2

    mesh = plsc.VectorSubcoreMesh(core_axis_name="c", subcore_axis_name="s")

    @functools.partial(
        pl.kernel,
        mesh=mesh,
        compiler_params=pltpu.CompilerParams(needs_layout_passes=False,
                                             use_tc_tiling_on_sc=False),
        out_type=jax.ShapeDtypeStruct((n_tokens, HIDDEN), jnp.float32),
        scratch_types=[
            pltpu.VMEM((chunks_per_w, CHUNK), jnp.int32),
            pltpu.VMEM((tok_per_w + L,), jnp.int32),
            pltpu.VMEM((2 * SEQ, HIDDEN), jnp.float32),
            pltpu.VMEM((2, HIDDEN), jnp.float32),
            pltpu.VMEM((HIDDEN,), jnp.float32),
            pltpu.VMEM((HIDDEN,), jnp.float32),
            pltpu.VMEM((CHUNK, HIDDEN), jnp.float32),
            pltpu.VMEM((CHUNK, HIDDEN), jnp.float32),
            pltpu.VMEM((CHUNK, HIDDEN), jnp.float32),
            pltpu.VMEM((CHUNK, HIDDEN), jnp.float32),
            pltpu.SemaphoreType.DMA,
            pltpu.SemaphoreType.DMA,
            pltpu.SemaphoreType.DMA,
            pltpu.SemaphoreType.DMA,
        ],
    )
    def sc_kernel(ids_hbm, tt_hbm, word_hbm, pos_hbm, type_hbm, gamma_hbm,
                  beta_hbm, out_hbm, ids_v, tt_v, post_v, type_v, gamma_v,
                  beta_v, gbuf0, gbuf1, obuf0, obuf1, sem_g0, sem_g1, sem_w0,
                  sem_w1):
        gb = [gbuf0, gbuf1]
        ob = [obuf0, obuf1]
        sg = [sem_g0, sem_g1]
        sw = [sem_w0, sem_w1]

        wid = lax.axis_index("s") * 2 + lax.axis_index("c")
        tok_base = wid * tok_per_w
        row_base = wid * chunks_per_w

        pltpu.sync_copy(pos_hbm.at[pl.ds(0, SEQ)], post_v.at[pl.ds(0, SEQ)])
        pltpu.sync_copy(pos_hbm.at[pl.ds(0, SEQ)], post_v.at[pl.ds(SEQ, SEQ)])
        pltpu.sync_copy(type_hbm, type_v)
        pltpu.sync_copy(gamma_hbm, gamma_v)
        pltpu.sync_copy(beta_hbm, beta_v)
        pltpu.sync_copy(ids_hbm.at[pl.ds(row_base, chunks_per_w)], ids_v)
        pltpu.sync_copy(tt_hbm.at[pl.ds(tok_base, tok_per_w)],
                        tt_v.at[pl.ds(0, tok_per_w)])

        g = [gamma_v[pl.ds(s * L, L)] for s in range(NSEG)]
        bt = [beta_v[pl.ds(s * L, L)] for s in range(NSEG)]
        t0 = [type_v[0, pl.ds(s * L, L)] for s in range(NSEG)]
        t1 = [type_v[1, pl.ds(s * L, L)] for s in range(NSEG)]

        def fold_body(r, carry):
            for s in range(NSEG):
                ds = pl.ds(s * L, L)
                post_v[r, ds] = post_v[r, ds] + t0[s]
                post_v[SEQ + r, ds] = post_v[SEQ + r, ds] + t1[s]
            return carry

        lax.fori_loop(0, SEQ, fold_body, 0, unroll=2)

        def gather_wait(b):
            pltpu.make_async_copy(word_hbm.at[ids_v.at[0]], gb[b],
                                  sg[b]).wait()

        def write_wait(b):
            pltpu.make_async_copy(ob[b], out_hbm.at[pl.ds(0, CHUNK)],
                                  sw[b]).wait()

        pltpu.async_copy(word_hbm.at[ids_v.at[0]], gb[0], sg[0])

        def group_body(grp, carry):
            for b in range(2):
                q = grp * 2 + b

                def prefetch():
                    pltpu.async_copy(word_hbm.at[ids_v.at[q + 1]], gb[1 - b],
                                     sg[1 - b])

                if b == 0:
                    prefetch()
                else:
                    pl.when(grp < groups - 1)(prefetch)

                pl.when(grp >= 1)(lambda: write_wait(b))
                gather_wait(b)

                gbuf = gb[b]
                obuf = ob[b]
                pos_base = b * CHUNK
                ttq_base = q * CHUNK

                def tok_body(i, tcarry):
                    tt = tt_v[pl.ds(ttq_base + i, L)][0]
                    prow = tt * SEQ + pos_base + i
                    xs = []
                    acc_s = None
                    acc_q = None
                    for s in range(NSEG):
                        ds = pl.ds(s * L, L)
                        x = gbuf[i, ds] + post_v[prow, ds]
                        xs.append(x)
                        acc_s = x if acc_s is None else acc_s + x
                        acc_q = x * x if acc_q is None else acc_q + x * x
                    mean = jnp.sum(acc_s) * (1.0 / HIDDEN)
                    var = jnp.sum(acc_q) * (1.0 / HIDDEN) - mean * mean
                    xh = jnp.full((L,), 0.5 * var + 0.5e-5)
                    yi = 0x5F3759DF - (plsc.bitcast(
                        jnp.full((L,), var + 1e-5), jnp.int32) >> 1)
                    y = plsc.bitcast(yi, jnp.float32)
                    for _ in range(2):
                        y = y * (1.5 - xh * y * y)
                    mean_v = jnp.full((L,), mean)
                    for s in range(NSEG):
                        obuf[i, pl.ds(s * L, L)] = ((xs[s] - mean_v) * y *
                                                    g[s] + bt[s])
                    return tcarry

                lax.fori_loop(0, CHUNK, tok_body, 0, unroll=2)

                pltpu.async_copy(
                    obuf, out_hbm.at[pl.ds(tok_base + q * CHUNK, CHUNK)],
                    sw[b])
            return carry

        lax.fori_loop(0, groups, group_body, 0)
        write_wait(0)
        write_wait(1)

    return sc_kernel


def kernel(input_ids, token_type_ids, word_emb, pos_emb, type_emb, ln_gamma,
           ln_beta):
    batch, seq = input_ids.shape
    assert seq == SEQ
    ids = input_ids.astype(jnp.int32).reshape(-1, CHUNK)
    tt = token_type_ids.astype(jnp.int32).reshape(-1)
    out = _make_sc_kernel(batch)(ids, tt, word_emb, pos_emb, type_emb,
                                 ln_gamma, ln_beta)
    return out.reshape(batch, seq, HIDDEN)


# speedup vs baseline: 1.1367x; 1.1367x over previous
import functools

import jax
import jax.numpy as jnp
from jax import lax
from jax.experimental import pallas as pl
from jax.experimental.pallas import tpu as pltpu
from jax.experimental.pallas import tpu_sc as plsc

HIDDEN = 128
SEQ = 200
L = 16
NSEG = HIDDEN // L
NW = 32
CHUNK = 100


@functools.lru_cache(maxsize=None)
def _make_sc_kernel(batch: int):
    n_tokens = batch * SEQ
    assert n_tokens % (NW * SEQ) == 0
    tok_per_w = n_tokens // NW
    chunks_per_w = tok_per_w // CHUNK
    assert chunks_per_w % 2 == 0
    groups = chunks_per_w // 2

    mesh = plsc.VectorSubcoreMesh(core_axis_name="c", subcore_axis_name="s")

    @functools.partial(
        pl.kernel,
        mesh=mesh,
        compiler_params=pltpu.CompilerParams(needs_layout_passes=False,
                                             use_tc_tiling_on_sc=False),
        out_type=jax.ShapeDtypeStruct((n_tokens, HIDDEN), jnp.float32),
        scratch_types=[
            pltpu.VMEM((chunks_per_w, CHUNK), jnp.int32),
            pltpu.VMEM((tok_per_w + L,), jnp.float32),
            pltpu.VMEM((SEQ, HIDDEN), jnp.float32),
            pltpu.VMEM((2, HIDDEN), jnp.float32),
            pltpu.VMEM((HIDDEN,), jnp.float32),
            pltpu.VMEM((HIDDEN,), jnp.float32),
            pltpu.VMEM((CHUNK, HIDDEN), jnp.float32),
            pltpu.VMEM((CHUNK, HIDDEN), jnp.float32),
            pltpu.VMEM((CHUNK, HIDDEN), jnp.float32),
            pltpu.VMEM((CHUNK, HIDDEN), jnp.float32),
            pltpu.SemaphoreType.DMA,
            pltpu.SemaphoreType.DMA,
            pltpu.SemaphoreType.DMA,
            pltpu.SemaphoreType.DMA,
        ],
    )
    def sc_kernel(ids_hbm, ttf_hbm, word_hbm, pos_hbm, type_hbm, gamma_hbm,
                  beta_hbm, out_hbm, ids_v, ttf_v, post0_v, type_v, gamma_v,
                  beta_v, gbuf0, gbuf1, obuf0, obuf1, sem_g0, sem_g1, sem_w0,
                  sem_w1):
        gb = [gbuf0, gbuf1]
        ob = [obuf0, obuf1]
        sg = [sem_g0, sem_g1]
        sw = [sem_w0, sem_w1]

        wid = lax.axis_index("s") * 2 + lax.axis_index("c")
        tok_base = wid * tok_per_w
        row_base = wid * chunks_per_w

        pltpu.sync_copy(pos_hbm.at[pl.ds(0, SEQ)], post0_v)
        pltpu.sync_copy(type_hbm, type_v)
        pltpu.sync_copy(gamma_hbm, gamma_v)
        pltpu.sync_copy(beta_hbm, beta_v)
        pltpu.sync_copy(ids_hbm.at[pl.ds(row_base, chunks_per_w)], ids_v)
        pltpu.sync_copy(ttf_hbm.at[pl.ds(tok_base, tok_per_w)],
                        ttf_v.at[pl.ds(0, tok_per_w)])

        g = [gamma_v[pl.ds(s * L, L)] for s in range(NSEG)]
        bt = [beta_v[pl.ds(s * L, L)] for s in range(NSEG)]
        t0 = [type_v[0, pl.ds(s * L, L)] for s in range(NSEG)]
        td = [type_v[1, pl.ds(s * L, L)] - t0[s] for s in range(NSEG)]

        e0 = (lax.iota(jnp.int32, L) == 0).astype(jnp.float32)

        def bcast0(v):
            return plsc.cumsum(v * e0)

        def bcast_total(v):
            return bcast0(jnp.flip(plsc.cumsum(v), 0))

        def fold_body(r, carry):
            for s in range(NSEG):
                ds = pl.ds(s * L, L)
                post0_v[r, ds] = post0_v[r, ds] + t0[s]
            return carry

        lax.fori_loop(0, SEQ, fold_body, 0, unroll=2)

        def gather_wait(b):
            pltpu.make_async_copy(word_hbm.at[ids_v.at[0]], gb[b],
                                  sg[b]).wait()

        def write_wait(b):
            pltpu.make_async_copy(ob[b], out_hbm.at[pl.ds(0, CHUNK)],
                                  sw[b]).wait()

        pltpu.async_copy(word_hbm.at[ids_v.at[0]], gb[0], sg[0])

        def group_body(grp, carry):
            for b in range(2):
                q = grp * 2 + b

                def prefetch():
                    pltpu.async_copy(word_hbm.at[ids_v.at[q + 1]], gb[1 - b],
                                     sg[1 - b])

                if b == 0:
                    prefetch()
                else:
                    pl.when(grp < groups - 1)(prefetch)

                pl.when(grp >= 1)(lambda: write_wait(b))
                gather_wait(b)

                gbuf = gb[b]
                obuf = ob[b]
                pos_base = b * CHUNK
                ttq_base = q * CHUNK

                def tok_body(i, tcarry):
                    ttf = bcast0(ttf_v[pl.ds(ttq_base + i, L)])
                    xs = []
                    acc_s = None
                    acc_q = None
                    for s in range(NSEG):
                        ds = pl.ds(s * L, L)
                        x = (gbuf[i, ds] + post0_v[pos_base + i, ds] +
                             ttf * td[s])
                        xs.append(x)
                        acc_s = x if acc_s is None else acc_s + x
                        acc_q = x * x if acc_q is None else acc_q + x * x
                    mean = bcast_total(acc_s) * (1.0 / HIDDEN)
                    sq = bcast_total(acc_q) * (1.0 / HIDDEN)
                    var = sq - mean * mean
                    xv = var + 1e-5
                    xh = 0.5 * xv
                    yi = 0x5F3759DF - (plsc.bitcast(xv, jnp.int32) >> 1)
                    y = plsc.bitcast(yi, jnp.float32)
                    for _ in range(2):
                        y = y * (1.5 - xh * y * y)
                    for s in range(NSEG):
                        obuf[i, pl.ds(s * L, L)] = ((xs[s] - mean) * y *
                                                    g[s] + bt[s])
                    return tcarry

                lax.fori_loop(0, CHUNK, tok_body, 0, unroll=2)

                pltpu.async_copy(
                    obuf, out_hbm.at[pl.ds(tok_base + q * CHUNK, CHUNK)],
                    sw[b])
            return carry

        lax.fori_loop(0, groups, group_body, 0)
        write_wait(0)
        write_wait(1)

    return sc_kernel


def kernel(input_ids, token_type_ids, word_emb, pos_emb, type_emb, ln_gamma,
           ln_beta):
    batch, seq = input_ids.shape
    assert seq == SEQ
    ids = input_ids.astype(jnp.int32).reshape(-1, CHUNK)
    ttf = token_type_ids.astype(jnp.float32).reshape(-1)
    out = _make_sc_kernel(batch)(ids, ttf, word_emb, pos_emb, type_emb,
                                 ln_gamma, ln_beta)
    return out.reshape(batch, seq, HIDDEN)
